# baseline (speedup 1.0000x reference)
import functools

import jax
import jax.numpy as jnp
from jax import lax
from jax.experimental import pallas as pl
from jax.experimental.pallas import tpu as pltpu
from jax.experimental.pallas import tpu_sc as plsc

N = 10000
D = 128
E = 160000
R = 3
NC = 2
NS = 16
CHUNK = 128
KMAX = 40
EP = NC * NS * KMAX * CHUNK
NCHUNK = EP // CHUNK
NP = 10240
ROWS_PER_SUB = NP // NS
ZROWS = 32


def _sc_body(feat_ref, edges_ref, out_ref, acc, zbuf, gbuf0, gbuf1,
             sidx0, sidx1, didx0, didx1, sem0, sem1):
    c = lax.axis_index("c")
    s = lax.axis_index("s")
    sidxs = (sidx0, sidx1)
    didxs = (didx0, didx1)

    @pl.loop(0, ZROWS)
    def _zero_zbuf(i):
        for jj in range(D // 16):
            zbuf[i, pl.ds(jj * 16, 16)] = jnp.zeros((16,), jnp.float32)

    row0 = s * ROWS_PER_SUB
    base = c * (NS * KMAX)
    gbufs = (gbuf0, gbuf1)
    sems = (sem0, sem1)
    for r in range(R):
        for z in range(ROWS_PER_SUB // ZROWS):
            pltpu.sync_copy(zbuf, acc.at[pl.ds(row0 + z * ZROWS, ZROWS)])
        plsc.subcore_barrier()

        @pl.loop(0, KMAX)
        def _chunks(k):
            j = base + k * NS + s
            pltpu.sync_copy(edges_ref.at[r, 0, j], sidx0)
            pltpu.sync_copy(edges_ref.at[r, 1, j], didx0)
            pltpu.sync_copy(feat_ref.at[sidx0], gbuf0)
            pltpu.sync_copy(gbuf0, acc.at[didx0], add=True)

        plsc.subcore_barrier()

        pltpu.sync_copy(acc.at[pl.ds(row0, ROWS_PER_SUB)],
                        out_ref.at[r * NC + c, pl.ds(row0, ROWS_PER_SUB)])
        plsc.subcore_barrier()


_sc_aggregate = functools.partial(
    pl.kernel,
    out_type=jax.ShapeDtypeStruct((R * NC, NP, D), jnp.float32),
    mesh=plsc.VectorSubcoreMesh(
        core_axis_name="c", subcore_axis_name="s",
        num_cores=NC, num_subcores=NS),
    scratch_types=[
        pltpu.VMEM_SHARED((NP, D), jnp.float32),
        pltpu.VMEM((ZROWS, D), jnp.float32),
        pltpu.VMEM((CHUNK, D), jnp.float32),
        pltpu.VMEM((CHUNK, D), jnp.float32),
        pltpu.VMEM((CHUNK,), jnp.int32),
        pltpu.VMEM((CHUNK,), jnp.int32),
        pltpu.VMEM((CHUNK,), jnp.int32),
        pltpu.VMEM((CHUNK,), jnp.int32),
        pltpu.SemaphoreType.DMA,
        pltpu.SemaphoreType.DMA,
    ],
)(_sc_body)


BLK = 1000


def _tc_body(parts_ref, feat_ref, wt_ref, out_ref):
    q0 = parts_ref[0] + parts_ref[1]
    q1 = parts_ref[2] + parts_ref[3]
    q2 = parts_ref[4] + parts_ref[5]
    h = jnp.dot(feat_ref[...], wt_ref[3], preferred_element_type=jnp.float32)
    h = h + jnp.dot(q0, wt_ref[0], preferred_element_type=jnp.float32)
    h = h + jnp.dot(q1, wt_ref[1], preferred_element_type=jnp.float32)
    h = h - jnp.dot(q2, wt_ref[2], preferred_element_type=jnp.float32)
    out_ref[...] = jnp.maximum(h, 0.0)


def _tc_combine(parts, feats, wt):
    return pl.pallas_call(
        _tc_body,
        grid=(N // BLK,),
        in_specs=[
            pl.BlockSpec((R * NC, BLK, D), lambda i: (0, i, 0)),
            pl.BlockSpec((BLK, D), lambda i: (i, 0)),
            pl.BlockSpec((4, D, D), lambda i: (0, 0, 0)),
        ],
        out_specs=pl.BlockSpec((BLK, D), lambda i: (i, 0)),
        out_shape=jax.ShapeDtypeStruct((N, D), jnp.float32),
    )(parts, feats, wt)


def kernel(features, W_r0, W_r1, W_r2, W_self, edge_index_r0, edge_index_r1,
           edge_index_r2):
    edges = jnp.stack([edge_index_r0, edge_index_r1, edge_index_r2])
    npad = EP - E
    pad_src = jnp.zeros((R, npad), jnp.int32)
    pad_dst = N + (jnp.arange(npad, dtype=jnp.int32) % (NP - N))
    pad = jnp.stack([pad_src, jnp.broadcast_to(pad_dst, (R, npad))], axis=1)
    edges_p = jnp.concatenate([edges, pad], axis=2).reshape(R, 2, NCHUNK, CHUNK)
    parts = _sc_aggregate(features, edges_p)
    wt = jnp.stack([W_r0.T, W_r1.T, W_r2.T, W_self.T])
    return _tc_combine(parts, features, wt)


# speedup vs baseline: 1.0031x; 1.0031x over previous
import functools

import jax
import jax.numpy as jnp
from jax import lax
from jax.experimental import pallas as pl
from jax.experimental.pallas import tpu as pltpu
from jax.experimental.pallas import tpu_sc as plsc

N = 10000
D = 128
E = 160000
R = 3
NC = 2
NS = 16
CHUNK = 128
KMAX = 40
EP = NC * NS * KMAX * CHUNK
NCHUNK = EP // CHUNK
NP = 10240
ROWS_PER_SUB = NP // NS
ZROWS = 128


def _sc_body(feat_ref, edges_ref, out_ref, acc, zbuf, gbuf0, sidx0, didx0):
    c = lax.axis_index("c")
    s = lax.axis_index("s")

    @pl.loop(0, ZROWS)
    def _zero_zbuf(i):
        for jj in range(D // 16):
            zbuf[i, pl.ds(jj * 16, 16)] = jnp.zeros((16,), jnp.float32)

    row0 = s * ROWS_PER_SUB
    base = c * (NS * KMAX)
    for r in range(R):
        for z in range(ROWS_PER_SUB // ZROWS):
            pltpu.sync_copy(zbuf, acc.at[pl.ds(row0 + z * ZROWS, ZROWS)])
        plsc.subcore_barrier()

        @pl.loop(0, KMAX)
        def _chunks(k):
            j = base + k * NS + s
            pltpu.sync_copy(edges_ref.at[r, 0, j], sidx0)
            pltpu.sync_copy(edges_ref.at[r, 1, j], didx0)
            pltpu.sync_copy(feat_ref.at[sidx0], gbuf0)
            pltpu.sync_copy(gbuf0, acc.at[didx0], add=True)

        plsc.subcore_barrier()

        pltpu.sync_copy(acc.at[pl.ds(row0, ROWS_PER_SUB)],
                        out_ref.at[r * NC + c, pl.ds(row0, ROWS_PER_SUB)])
        plsc.subcore_barrier()


_sc_aggregate = functools.partial(
    pl.kernel,
    out_type=jax.ShapeDtypeStruct((R * NC, NP, D), jnp.float32),
    mesh=plsc.VectorSubcoreMesh(
        core_axis_name="c", subcore_axis_name="s",
        num_cores=NC, num_subcores=NS),
    scratch_types=[
        pltpu.VMEM_SHARED((NP, D), jnp.float32),
        pltpu.VMEM((ZROWS, D), jnp.float32),
        pltpu.VMEM((CHUNK, D), jnp.float32),
        pltpu.VMEM((CHUNK,), jnp.int32),
        pltpu.VMEM((CHUNK,), jnp.int32),
    ],
)(_sc_body)


BLK = 1000


def _tc_body(parts_ref, feat_ref, wt_ref, out_ref):
    q0 = parts_ref[0] + parts_ref[1]
    q1 = parts_ref[2] + parts_ref[3]
    q2 = parts_ref[4] + parts_ref[5]
    h = jnp.dot(feat_ref[...], wt_ref[3], preferred_element_type=jnp.float32)
    h = h + jnp.dot(q0, wt_ref[0], preferred_element_type=jnp.float32)
    h = h + jnp.dot(q1, wt_ref[1], preferred_element_type=jnp.float32)
    h = h - jnp.dot(q2, wt_ref[2], preferred_element_type=jnp.float32)
    out_ref[...] = jnp.maximum(h, 0.0)


def _tc_combine(parts, feats, wt):
    return pl.pallas_call(
        _tc_body,
        grid=(N // BLK,),
        in_specs=[
            pl.BlockSpec((R * NC, BLK, D), lambda i: (0, i, 0)),
            pl.BlockSpec((BLK, D), lambda i: (i, 0)),
            pl.BlockSpec((4, D, D), lambda i: (0, 0, 0)),
        ],
        out_specs=pl.BlockSpec((BLK, D), lambda i: (i, 0)),
        out_shape=jax.ShapeDtypeStruct((N, D), jnp.float32),
    )(parts, feats, wt)


def kernel(features, W_r0, W_r1, W_r2, W_self, edge_index_r0, edge_index_r1,
           edge_index_r2):
    edges = jnp.stack([edge_index_r0, edge_index_r1, edge_index_r2])
    npad = EP - E
    pad_src = jnp.zeros((R, npad), jnp.int32)
    pad_dst = N + (jnp.arange(npad, dtype=jnp.int32) % (NP - N))
    pad = jnp.stack([pad_src, jnp.broadcast_to(pad_dst, (R, npad))], axis=1)
    edges_p = jnp.concatenate([edges, pad], axis=2).reshape(R, 2, NCHUNK, CHUNK)
    parts = _sc_aggregate(features, edges_p)
    wt = jnp.stack([W_r0.T, W_r1.T, W_r2.T, W_self.T])
    return _tc_combine(parts, features, wt)


# speedup vs baseline: 1.0096x; 1.0065x over previous
import functools

import jax
import jax.numpy as jnp
from jax import lax
from jax.experimental import pallas as pl
from jax.experimental.pallas import tpu as pltpu
from jax.experimental.pallas import tpu_sc as plsc

N = 10000
D = 128
E = 160000
R = 3
NC = 2
NS = 16
CHUNK = 128
KMAX = 40
EP = NC * NS * KMAX * CHUNK
NCHUNK = EP // CHUNK
NP = 10240
ROWS_PER_SUB = NP // NS
ZROWS = 128


def _sc_body(feat_ref, edges_ref, out_ref, acc, zbuf, gbuf0, sidx0, didx0):
    c = lax.axis_index("c")
    s = lax.axis_index("s")

    @pl.loop(0, ZROWS)
    def _zero_zbuf(i):
        for jj in range(D // 16):
            zbuf[i, pl.ds(jj * 16, 16)] = jnp.zeros((16,), jnp.float32)

    row0 = s * ROWS_PER_SUB
    base = c * (NS * KMAX)
    for r in range(R):
        for z in range(ROWS_PER_SUB // ZROWS):
            pltpu.sync_copy(zbuf, acc.at[pl.ds(row0 + z * ZROWS, ZROWS)])
        plsc.subcore_barrier()

        @pl.loop(0, KMAX)
        def _chunks(k):
            eb = (base + k * NS + s) * CHUNK
            pltpu.sync_copy(edges_ref.at[r, 0, pl.ds(eb, CHUNK)], sidx0)
            pltpu.sync_copy(edges_ref.at[r, 1, pl.ds(eb, CHUNK)], didx0)
            pltpu.sync_copy(feat_ref.at[sidx0], gbuf0)
            pltpu.sync_copy(gbuf0, acc.at[didx0], add=True)

        plsc.subcore_barrier()

        pltpu.sync_copy(acc.at[pl.ds(row0, ROWS_PER_SUB)],
                        out_ref.at[r * NC + c, pl.ds(row0, ROWS_PER_SUB)])
        plsc.subcore_barrier()


_sc_aggregate = functools.partial(
    pl.kernel,
    out_type=jax.ShapeDtypeStruct((R * NC, NP, D), jnp.float32),
    mesh=plsc.VectorSubcoreMesh(
        core_axis_name="c", subcore_axis_name="s",
        num_cores=NC, num_subcores=NS),
    scratch_types=[
        pltpu.VMEM_SHARED((NP, D), jnp.float32),
        pltpu.VMEM((ZROWS, D), jnp.float32),
        pltpu.VMEM((CHUNK, D), jnp.float32),
        pltpu.VMEM((CHUNK,), jnp.int32),
        pltpu.VMEM((CHUNK,), jnp.int32),
    ],
)(_sc_body)


BLK = 1000


def _tc_body(parts_ref, feat_ref, wt_ref, out_ref):
    q0 = parts_ref[0] + parts_ref[1]
    q1 = parts_ref[2] + parts_ref[3]
    q2 = parts_ref[4] + parts_ref[5]
    h = jnp.dot(feat_ref[...], wt_ref[3], preferred_element_type=jnp.float32)
    h = h + jnp.dot(q0, wt_ref[0], preferred_element_type=jnp.float32)
    h = h + jnp.dot(q1, wt_ref[1], preferred_element_type=jnp.float32)
    h = h - jnp.dot(q2, wt_ref[2], preferred_element_type=jnp.float32)
    out_ref[...] = jnp.maximum(h, 0.0)


def _tc_combine(parts, feats, wt):
    return pl.pallas_call(
        _tc_body,
        grid=(N // BLK,),
        in_specs=[
            pl.BlockSpec((R * NC, BLK, D), lambda i: (0, i, 0)),
            pl.BlockSpec((BLK, D), lambda i: (i, 0)),
            pl.BlockSpec((4, D, D), lambda i: (0, 0, 0)),
        ],
        out_specs=pl.BlockSpec((BLK, D), lambda i: (i, 0)),
        out_shape=jax.ShapeDtypeStruct((N, D), jnp.float32),
    )(parts, feats, wt)


def kernel(features, W_r0, W_r1, W_r2, W_self, edge_index_r0, edge_index_r1,
           edge_index_r2):
    edges = jnp.stack([edge_index_r0, edge_index_r1, edge_index_r2])
    npad = EP - E
    pad_src = jnp.zeros((R, npad), jnp.int32)
    pad_dst = N + (jnp.arange(npad, dtype=jnp.int32) % (NP - N))
    pad = jnp.stack([pad_src, jnp.broadcast_to(pad_dst, (R, npad))], axis=1)
    edges_p = jnp.concatenate([edges, pad], axis=2)
    parts = _sc_aggregate(features, edges_p)
    wt = jnp.stack([W_r0.T, W_r1.T, W_r2.T, W_self.T])
    return _tc_combine(parts, features, wt)


# speedup vs baseline: 2.0442x; 2.0249x over previous
import functools

import jax
import jax.numpy as jnp
from jax import lax
from jax.experimental import pallas as pl
from jax.experimental.pallas import tpu as pltpu
from jax.experimental.pallas import tpu_sc as plsc

N = 10000
D = 128
E = 160000
R = 3
NC = 2
NS = 16
CHUNK = 128
EPC = E // NC
CPC = EPC // CHUNK
KMAX = -(-CPC // NS)
NP = 10240
ROWS_PER_SUB = NP // NS
ZROWS = 128


def _sc_body(feat_ref, edges_ref, out_ref, acc, zbuf, gbuf, src_idx, dst_idx):
    c = lax.axis_index("c")
    s = lax.axis_index("s")

    @pl.loop(0, ZROWS)
    def _zero_zbuf(i):
        for jj in range(D // 16):
            zbuf[i, pl.ds(jj * 16, 16)] = jnp.zeros((16,), jnp.float32)

    row0 = s * ROWS_PER_SUB
    for r in range(R):
        for z in range(ROWS_PER_SUB // ZROWS):
            pltpu.sync_copy(zbuf, acc.at[pl.ds(row0 + z * ZROWS, ZROWS)])
        plsc.subcore_barrier()

        @pl.loop(0, KMAX)
        def _chunks(k):
            j = k * NS + s

            @pl.when(j < CPC)
            def _():
                base = c * EPC + j * CHUNK
                pltpu.sync_copy(edges_ref.at[r, 0, pl.ds(base, CHUNK)], src_idx)
                pltpu.sync_copy(edges_ref.at[r, 1, pl.ds(base, CHUNK)], dst_idx)
                pltpu.sync_copy(feat_ref.at[src_idx], gbuf)
                pltpu.sync_copy(gbuf, acc.at[dst_idx], add=True)

        plsc.subcore_barrier()

        pltpu.sync_copy(acc.at[pl.ds(row0, ROWS_PER_SUB)],
                        out_ref.at[r * NC + c, pl.ds(row0, ROWS_PER_SUB)])
        plsc.subcore_barrier()


_sc_aggregate = functools.partial(
    pl.kernel,
    out_type=jax.ShapeDtypeStruct((R * NC, NP, D), jnp.float32),
    mesh=plsc.VectorSubcoreMesh(
        core_axis_name="c", subcore_axis_name="s",
        num_cores=NC, num_subcores=NS),
    scratch_types=[
        pltpu.VMEM_SHARED((NP, D), jnp.float32),
        pltpu.VMEM((ZROWS, D), jnp.float32),
        pltpu.VMEM((CHUNK, D), jnp.float32),
        pltpu.VMEM((CHUNK,), jnp.int32),
        pltpu.VMEM((CHUNK,), jnp.int32),
    ],
)(_sc_body)


BLK = 1000


def _tc_body(parts_ref, feat_ref, wt_ref, out_ref):
    q0 = parts_ref[0] + parts_ref[1]
    q1 = parts_ref[2] + parts_ref[3]
    q2 = parts_ref[4] + parts_ref[5]
    h = jnp.dot(feat_ref[...], wt_ref[3], preferred_element_type=jnp.float32)
    h = h + jnp.dot(q0, wt_ref[0], preferred_element_type=jnp.float32)
    h = h + jnp.dot(q1, wt_ref[1], preferred_element_type=jnp.float32)
    h = h - jnp.dot(q2, wt_ref[2], preferred_element_type=jnp.float32)
    out_ref[...] = jnp.maximum(h, 0.0)


def _tc_combine(parts, feats, wt):
    return pl.pallas_call(
        _tc_body,
        grid=(N // BLK,),
        in_specs=[
            pl.BlockSpec((R * NC, BLK, D), lambda i: (0, i, 0)),
            pl.BlockSpec((BLK, D), lambda i: (i, 0)),
            pl.BlockSpec((4, D, D), lambda i: (0, 0, 0)),
        ],
        out_specs=pl.BlockSpec((BLK, D), lambda i: (i, 0)),
        out_shape=jax.ShapeDtypeStruct((N, D), jnp.float32),
    )(parts, feats, wt)


def kernel(features, W_r0, W_r1, W_r2, W_self, edge_index_r0, edge_index_r1,
           edge_index_r2):
    edges = jnp.stack([edge_index_r0, edge_index_r1, edge_index_r2])
    parts = _sc_aggregate(features, edges)
    wt = jnp.stack([W_r0.T, W_r1.T, W_r2.T, W_self.T])
    return _tc_combine(parts, features, wt)
